# baseline (speedup 1.0000x reference)
import dataclasses

import jax
import jax.numpy as jnp
from jax import lax
from jax.experimental import pallas as pl
from jax.experimental.pallas import tpu as pltpu
from jax.experimental.pallas import tpu_sc as plsc

OUT_SIZE = 16384
NCORES = 2
NSUB = 16
LANES = 16
EBLK = 128
SROWS = 72
NRING = 4
WCHUNK = 64
NCHUNK = 4


def _sc_spmm(x4, ii2d, io2d, v1d, *, rows_per_tile, in_size):
    mesh = plsc.VectorSubcoreMesh(core_axis_name="c", subcore_axis_name="s")
    out_type = jax.ShapeDtypeStruct((NCHUNK * OUT_SIZE, WCHUNK), jnp.bfloat16)
    rows_out = OUT_SIZE // NSUB
    rows_in = in_size // NSUB

    stages = []
    r = 0
    while r < rows_per_tile:
        n = min(SROWS, rows_per_tile - r)
        assert n % NRING == 0 and n % 8 == 0
        stages.append((r, n))
        r += n

    def body(x_hbm, ii_hbm, io_hbm, v_hbm, out_hbm,
             acc, x_sp, ii_v, io_v, v_v, rb0, rb1, rb2, rb3, gsem, ssem):
        c = lax.axis_index("c")
        s = lax.axis_index("s")
        row0 = s * rows_per_tile
        rbufs = [rb0, rb1, rb2, rb3]

        zero32 = jnp.zeros((2 * LANES,), jnp.bfloat16)

        def mul_block(rb, j):
            @plsc.parallel_loop(0, EBLK, step=LANES, unroll=2)
            def _(e0):
                v16 = v_v[pl.ds(j * EBLK + e0, LANES)]
                for t in range(LANES):
                    vspl = lax.gather(
                        v16, jnp.full((LANES, 1), t, jnp.int32),
                        lax.GatherDimensionNumbers(
                            offset_dims=(), collapsed_slice_dims=(0,),
                            start_index_map=(0,)),
                        slice_sizes=(1,),
                        mode=lax.GatherScatterMode.PROMISE_IN_BOUNDS)
                    vsplh = plsc.pack(vspl, vspl,
                                      format=plsc.PackFormat.INTERLEAVED)
                    for g in range(WCHUNK // (2 * LANES)):
                        sl = pl.ds(g * 2 * LANES, 2 * LANES)
                        rb[e0 + t, sl] = rb[e0 + t, sl] * vsplh

        for ci in range(NCHUNK // NCORES):
            chunk = c * (NCHUNK // NCORES) + ci

            pltpu.sync_copy(
                x_hbm.at[pl.ds(chunk * in_size + s * rows_in, rows_in)],
                x_sp.at[pl.ds(s * rows_in, rows_in)])

            @pl.loop(0, EBLK)
            def _(r):
                for k in range(WCHUNK // (2 * LANES)):
                    rb0[r, pl.ds(k * 2 * LANES, 2 * LANES)] = zero32

            @pl.loop(0, rows_out // EBLK)
            def _(k):
                pltpu.sync_copy(rb0, acc.at[pl.ds(s * rows_out + k * EBLK, EBLK)])
            plsc.subcore_barrier()

            for srow, slen in stages:
                brow = row0 + srow
                pltpu.sync_copy(ii_hbm.at[pl.ds(brow, slen)],
                                ii_v.at[pl.ds(0, slen)])
                pltpu.sync_copy(io_hbm.at[pl.ds(brow, slen)],
                                io_v.at[pl.ds(0, slen)])
                pltpu.sync_copy(v_hbm.at[pl.ds(brow * EBLK, slen * EBLK)],
                                v_v.at[pl.ds(0, slen * EBLK)])

                for q in range(NRING - 1):
                    pltpu.async_copy(x_sp.at[ii_v.at[q]], rbufs[q], gsem)

                @pl.loop(0, slen // NRING)
                def _(p):
                    for q in range(NRING):
                        j = p * NRING + q
                        rb = rbufs[q]
                        pltpu.make_async_copy(
                            x_sp.at[ii_v.at[j]], rb, gsem).wait()

                        @pl.when(j >= 1)
                        def _():
                            jm = j - 1
                            pltpu.make_async_copy(
                                rbufs[(q + NRING - 1) % NRING],
                                acc.at[io_v.at[jm]], ssem).wait()

                        @pl.when(j + NRING - 1 < slen)
                        def _():
                            jn = j + NRING - 1
                            pltpu.async_copy(
                                x_sp.at[ii_v.at[jn]],
                                rbufs[(q + NRING - 1) % NRING], gsem)

                        mul_block(rb, j)
                        pltpu.async_copy(rb, acc.at[io_v.at[j]], ssem,
                                         add=True)

                pltpu.make_async_copy(
                    rbufs[(slen - 1) % NRING],
                    acc.at[io_v.at[slen - 1]], ssem).wait()

            plsc.subcore_barrier()
            pltpu.sync_copy(
                acc.at[pl.ds(s * rows_out, rows_out)],
                out_hbm.at[pl.ds(chunk * OUT_SIZE + s * rows_out, rows_out)])
            plsc.subcore_barrier()

    cp = pltpu.CompilerParams()
    if "needs_layout_passes" in pltpu.CompilerParams.__dataclass_fields__:
        cp = dataclasses.replace(cp, needs_layout_passes=False)
    if "use_tc_tiling_on_sc" in pltpu.CompilerParams.__dataclass_fields__:
        cp = dataclasses.replace(cp, use_tc_tiling_on_sc=False)
    run = pl.kernel(
        body,
        out_type=out_type,
        mesh=mesh,
        compiler_params=cp,
        scratch_types=[
            pltpu.VMEM_SHARED((OUT_SIZE, WCHUNK), jnp.bfloat16),
            pltpu.VMEM_SHARED((in_size, WCHUNK), jnp.bfloat16),
            pltpu.VMEM((SROWS, EBLK), jnp.int32),
            pltpu.VMEM((SROWS, EBLK), jnp.int32),
            pltpu.VMEM((SROWS * EBLK,), jnp.float32),
            pltpu.VMEM((EBLK, WCHUNK), jnp.bfloat16),
            pltpu.VMEM((EBLK, WCHUNK), jnp.bfloat16),
            pltpu.VMEM((EBLK, WCHUNK), jnp.bfloat16),
            pltpu.VMEM((EBLK, WCHUNK), jnp.bfloat16),
            pltpu.SemaphoreType.DMA,
            pltpu.SemaphoreType.DMA,
        ],
    )
    return run(x4, ii2d, io2d, v1d)


@jax.jit
def kernel(x, v, indices_in, indices_out):
    batch, in_size = x.shape
    nnz = v.shape[0]
    assert batch == NCHUNK * WCHUNK

    per_tile = -(-nnz // (NSUB * EBLK * 8)) * EBLK * 8
    nnz_pad = per_tile * NSUB
    pad = nnz_pad - nnz
    ii = jnp.concatenate([indices_in, jnp.zeros((pad,), jnp.int32)])
    io = jnp.concatenate([indices_out, jnp.zeros((pad,), jnp.int32)])
    vp = jnp.concatenate([v, jnp.zeros((pad,), jnp.float32)])
    ii2d = ii.reshape(nnz_pad // EBLK, EBLK)
    io2d = io.reshape(nnz_pad // EBLK, EBLK)

    x4 = (x.reshape(NCHUNK, WCHUNK, in_size).transpose(0, 2, 1)
          .reshape(NCHUNK * in_size, WCHUNK).astype(jnp.bfloat16))

    yt4 = _sc_spmm(x4, ii2d, io2d, vp,
                   rows_per_tile=per_tile // EBLK, in_size=in_size)

    y = (yt4.reshape(NCHUNK, OUT_SIZE, WCHUNK).astype(jnp.float32)
         .transpose(0, 2, 1).reshape(batch, OUT_SIZE))
    return y


# speedup vs baseline: 1.0756x; 1.0756x over previous
import dataclasses

import jax
import jax.numpy as jnp
from jax import lax
from jax.experimental import pallas as pl
from jax.experimental.pallas import tpu as pltpu
from jax.experimental.pallas import tpu_sc as plsc

OUT_SIZE = 16384
NCORES = 2
NSUB = 16
LANES = 16
EBLK = 128
SROWS = 72
NRING = 8
WCHUNK = 64
NCHUNK = 4


def _sc_spmm(x4, ii2d, io2d, v1d, *, rows_per_tile, in_size):
    mesh = plsc.VectorSubcoreMesh(core_axis_name="c", subcore_axis_name="s")
    out_type = jax.ShapeDtypeStruct((NCHUNK * OUT_SIZE, WCHUNK), jnp.bfloat16)
    rows_out = OUT_SIZE // NSUB
    rows_in = in_size // NSUB

    stages = []
    r = 0
    while r < rows_per_tile:
        n = min(SROWS, rows_per_tile - r)
        assert n % NRING == 0 and n % 8 == 0
        stages.append((r, n))
        r += n

    def body(x_hbm, ii_hbm, io_hbm, v_hbm, out_hbm,
             acc, x_sp, ii_v, io_v, v_v,
             rb0, rb1, rb2, rb3, rb4, rb5, rb6, rb7, gsem, ssem):
        c = lax.axis_index("c")
        s = lax.axis_index("s")
        row0 = s * rows_per_tile
        rbufs = [rb0, rb1, rb2, rb3, rb4, rb5, rb6, rb7]

        zero32 = jnp.zeros((2 * LANES,), jnp.bfloat16)

        def mul_block(rb, j):
            @plsc.parallel_loop(0, EBLK, step=LANES, unroll=2)
            def _(e0):
                v16 = v_v[pl.ds(j * EBLK + e0, LANES)]
                for t in range(LANES):
                    vspl = lax.gather(
                        v16, jnp.full((LANES, 1), t, jnp.int32),
                        lax.GatherDimensionNumbers(
                            offset_dims=(), collapsed_slice_dims=(0,),
                            start_index_map=(0,)),
                        slice_sizes=(1,),
                        mode=lax.GatherScatterMode.PROMISE_IN_BOUNDS)
                    vsplh = plsc.pack(vspl, vspl,
                                      format=plsc.PackFormat.INTERLEAVED)
                    for g in range(WCHUNK // (2 * LANES)):
                        sl = pl.ds(g * 2 * LANES, 2 * LANES)
                        rb[e0 + t, sl] = rb[e0 + t, sl] * vsplh

        for ci in range(NCHUNK // NCORES):
            chunk = c * (NCHUNK // NCORES) + ci

            pltpu.sync_copy(
                x_hbm.at[pl.ds(chunk * in_size + s * rows_in, rows_in)],
                x_sp.at[pl.ds(s * rows_in, rows_in)])

            @pl.loop(0, EBLK)
            def _(r):
                for k in range(WCHUNK // (2 * LANES)):
                    rb0[r, pl.ds(k * 2 * LANES, 2 * LANES)] = zero32

            @pl.loop(0, rows_out // EBLK)
            def _(k):
                pltpu.sync_copy(rb0, acc.at[pl.ds(s * rows_out + k * EBLK, EBLK)])
            plsc.subcore_barrier()

            for srow, slen in stages:
                brow = row0 + srow
                pltpu.sync_copy(ii_hbm.at[pl.ds(brow, slen)],
                                ii_v.at[pl.ds(0, slen)])
                pltpu.sync_copy(io_hbm.at[pl.ds(brow, slen)],
                                io_v.at[pl.ds(0, slen)])
                pltpu.sync_copy(v_hbm.at[pl.ds(brow * EBLK, slen * EBLK)],
                                v_v.at[pl.ds(0, slen * EBLK)])

                for q in range(NRING - 1):
                    pltpu.async_copy(x_sp.at[ii_v.at[q]], rbufs[q], gsem)

                @pl.loop(0, slen // NRING)
                def _(p):
                    for q in range(NRING):
                        j = p * NRING + q
                        rb = rbufs[q]
                        pltpu.make_async_copy(
                            x_sp.at[ii_v.at[j]], rb, gsem).wait()

                        @pl.when(j >= 1)
                        def _():
                            jm = j - 1
                            pltpu.make_async_copy(
                                rbufs[(q + NRING - 1) % NRING],
                                acc.at[io_v.at[jm]], ssem).wait()

                        @pl.when(j + NRING - 1 < slen)
                        def _():
                            jn = j + NRING - 1
                            pltpu.async_copy(
                                x_sp.at[ii_v.at[jn]],
                                rbufs[(q + NRING - 1) % NRING], gsem)

                        mul_block(rb, j)
                        pltpu.async_copy(rb, acc.at[io_v.at[j]], ssem,
                                         add=True)

                pltpu.make_async_copy(
                    rbufs[(slen - 1) % NRING],
                    acc.at[io_v.at[slen - 1]], ssem).wait()

            plsc.subcore_barrier()
            pltpu.sync_copy(
                acc.at[pl.ds(s * rows_out, rows_out)],
                out_hbm.at[pl.ds(chunk * OUT_SIZE + s * rows_out, rows_out)])
            plsc.subcore_barrier()

    cp = pltpu.CompilerParams()
    if "needs_layout_passes" in pltpu.CompilerParams.__dataclass_fields__:
        cp = dataclasses.replace(cp, needs_layout_passes=False)
    if "use_tc_tiling_on_sc" in pltpu.CompilerParams.__dataclass_fields__:
        cp = dataclasses.replace(cp, use_tc_tiling_on_sc=False)
    run = pl.kernel(
        body,
        out_type=out_type,
        mesh=mesh,
        compiler_params=cp,
        scratch_types=[
            pltpu.VMEM_SHARED((OUT_SIZE, WCHUNK), jnp.bfloat16),
            pltpu.VMEM_SHARED((in_size, WCHUNK), jnp.bfloat16),
            pltpu.VMEM((SROWS, EBLK), jnp.int32),
            pltpu.VMEM((SROWS, EBLK), jnp.int32),
            pltpu.VMEM((SROWS * EBLK,), jnp.float32),
            pltpu.VMEM((EBLK, WCHUNK), jnp.bfloat16),
            pltpu.VMEM((EBLK, WCHUNK), jnp.bfloat16),
            pltpu.VMEM((EBLK, WCHUNK), jnp.bfloat16),
            pltpu.VMEM((EBLK, WCHUNK), jnp.bfloat16),
            pltpu.VMEM((EBLK, WCHUNK), jnp.bfloat16),
            pltpu.VMEM((EBLK, WCHUNK), jnp.bfloat16),
            pltpu.VMEM((EBLK, WCHUNK), jnp.bfloat16),
            pltpu.VMEM((EBLK, WCHUNK), jnp.bfloat16),
            pltpu.SemaphoreType.DMA,
            pltpu.SemaphoreType.DMA,
        ],
    )
    return run(x4, ii2d, io2d, v1d)


@jax.jit
def kernel(x, v, indices_in, indices_out):
    batch, in_size = x.shape
    nnz = v.shape[0]
    assert batch == NCHUNK * WCHUNK

    per_tile = -(-nnz // (NSUB * EBLK * 8)) * EBLK * 8
    nnz_pad = per_tile * NSUB
    pad = nnz_pad - nnz
    ii = jnp.concatenate([indices_in, jnp.zeros((pad,), jnp.int32)])
    io = jnp.concatenate([indices_out, jnp.zeros((pad,), jnp.int32)])
    vp = jnp.concatenate([v, jnp.zeros((pad,), jnp.float32)])
    ii2d = ii.reshape(nnz_pad // EBLK, EBLK)
    io2d = io.reshape(nnz_pad // EBLK, EBLK)

    x4 = (x.reshape(NCHUNK, WCHUNK, in_size).transpose(0, 2, 1)
          .reshape(NCHUNK * in_size, WCHUNK).astype(jnp.bfloat16))

    yt4 = _sc_spmm(x4, ii2d, io2d, vp,
                   rows_per_tile=per_tile // EBLK, in_size=in_size)

    y = (yt4.reshape(NCHUNK, OUT_SIZE, WCHUNK).astype(jnp.float32)
         .transpose(0, 2, 1).reshape(batch, OUT_SIZE))
    return y
